# baseline (speedup 1.0000x reference)
import functools

import jax
import jax.numpy as jnp
from jax import lax
from jax.experimental import pallas as pl
from jax.experimental.pallas import tpu as pltpu
from jax.experimental.pallas import tpu_sc as plsc

N_USER = 100000
N_ITEM = 100000
D = 64
B = 16384

NC = 2
NS = 16
NW = NC * NS
BPW = B // NW
CH = 128
NCH = BPW // CH
PAD = 16 * 17


@functools.partial(
    pl.kernel,
    out_type=(
        jax.ShapeDtypeStruct((B,), jnp.float32),
        jax.ShapeDtypeStruct((B,), jnp.float32),
    ),
    mesh=plsc.VectorSubcoreMesh(core_axis_name="c", subcore_axis_name="s"),
    compiler_params=pltpu.CompilerParams(needs_layout_passes=False),
    scratch_types=[
        pltpu.VMEM((BPW,), jnp.int32),
        pltpu.VMEM((BPW,), jnp.int32),
        pltpu.VMEM((BPW,), jnp.int32),
        pltpu.VMEM((BPW,), jnp.int32),
        pltpu.VMEM((BPW,), jnp.int32),
        pltpu.VMEM((BPW,), jnp.int32),
        pltpu.VMEM((CH, 2 * D), jnp.float32),
        pltpu.VMEM((CH, 2 * D), jnp.float32),
        pltpu.VMEM((CH, 2 * D), jnp.float32),
        pltpu.VMEM((BPW,), jnp.float32),
        pltpu.VMEM((BPW,), jnp.float32),
        pltpu.VMEM((4 * PAD,), jnp.float32),
        pltpu.VMEM((4 * PAD,), jnp.float32),
        pltpu.SemaphoreType.DMA,
        pltpu.SemaphoreType.DMA,
    ],
)
def _mcbpr_sc(u_hbm, i_hbm, j_hbm, eu_hbm, ei_hbm, oi_hbm, oj_hbm,
              u_v, i_v, j_v, uh_v, ih_v, jh_v, ur_v, ir_v, jr_v,
              oi_v, oj_v, pi_v, pj_v, sem, isem):
    wid = lax.axis_index("s") * NC + lax.axis_index("c")
    base = wid * BPW

    idx_copies = [
        pltpu.async_copy(u_hbm.at[pl.ds(base, BPW)], u_v, isem),
        pltpu.async_copy(i_hbm.at[pl.ds(base, BPW)], i_v, isem),
        pltpu.async_copy(j_hbm.at[pl.ds(base, BPW)], j_v, isem),
    ]
    for c in idx_copies:
        c.wait()

    def halve_body(k, carry):
        sl = pl.ds(k * 16, 16)
        uh_v[sl] = lax.shift_right_logical(u_v[sl], 1)
        ih_v[sl] = lax.shift_right_logical(i_v[sl], 1)
        jh_v[sl] = lax.shift_right_logical(j_v[sl], 1)
        return carry

    lax.fori_loop(0, BPW // 16, halve_body, 0)

    lanes = lax.iota(jnp.int32, 16)
    zero = jnp.zeros((16,), jnp.float32)
    one = jnp.ones((16,), jnp.int32)
    tidx = lanes * 17

    def chunk_body(c, carry):
        cbase = c * CH
        copies = [
            pltpu.async_copy(eu_hbm.at[uh_v.at[pl.ds(cbase, CH)]], ur_v, sem),
            pltpu.async_copy(ei_hbm.at[ih_v.at[pl.ds(cbase, CH)]], ir_v, sem),
            pltpu.async_copy(ei_hbm.at[jh_v.at[pl.ds(cbase, CH)]], jr_v, sem),
        ]
        for cp in copies:
            cp.wait()

        def group_body(g, gcarry):
            for r in range(16):
                row = g * 16 + r
                ul = [ur_v[row, pl.ds(16 * t, 16)] for t in range(4)]
                uh = [ur_v[row, pl.ds(64 + 16 * t, 16)] for t in range(4)]
                il = [ir_v[row, pl.ds(16 * t, 16)] for t in range(4)]
                ih = [ir_v[row, pl.ds(64 + 16 * t, 16)] for t in range(4)]
                jl = [jr_v[row, pl.ds(16 * t, 16)] for t in range(4)]
                jh = [jr_v[row, pl.ds(64 + 16 * t, 16)] for t in range(4)]

                def dot4(a, b):
                    return (a[0] * b[0] + a[1] * b[1]
                            + a[2] * b[2] + a[3] * b[3])

                pi_v[pl.ds(r * 17, 16)] = dot4(ul, il)
                pi_v[pl.ds(PAD + r * 17, 16)] = dot4(ul, ih)
                pi_v[pl.ds(2 * PAD + r * 17, 16)] = dot4(uh, il)
                pi_v[pl.ds(3 * PAD + r * 17, 16)] = dot4(uh, ih)
                pj_v[pl.ds(r * 17, 16)] = dot4(ul, jl)
                pj_v[pl.ds(PAD + r * 17, 16)] = dot4(ul, jh)
                pj_v[pl.ds(2 * PAD + r * 17, 16)] = dot4(uh, jl)
                pj_v[pl.ds(3 * PAD + r * 17, 16)] = dot4(uh, jh)
            gsl = pl.ds(cbase + g * 16, 16)
            pu = u_v[gsl] & one
            pii = i_v[gsl] & one
            pjj = j_v[gsl] & one
            bi = tidx + (2 * pu + pii) * PAD
            bj = tidx + (2 * pu + pjj) * PAD
            ai = zero
            aj = zero
            for col in range(16):
                ai = ai + plsc.load_gather(pi_v, [bi + col])
                aj = aj + plsc.load_gather(pj_v, [bj + col])
            oi_v[gsl] = ai
            oj_v[gsl] = aj
            return gcarry

        lax.fori_loop(0, CH // 16, group_body, 0)
        return carry

    lax.fori_loop(0, NCH, chunk_body, 0)

    pltpu.sync_copy(oi_v, oi_hbm.at[pl.ds(base, BPW)])
    pltpu.sync_copy(oj_v, oj_hbm.at[pl.ds(base, BPW)])


def kernel(u, i, j, embed_user, embed_item):
    lane = (u[0] * 0 + 1).astype(jnp.float32)
    return _mcbpr_sc(u.astype(jnp.int32), i.astype(jnp.int32),
                     j.astype(jnp.int32),
                     embed_user.reshape(N_USER // 2, 2 * D) * lane,
                     embed_item.reshape(N_ITEM // 2, 2 * D) * lane)


# speedup vs baseline: 1.1193x; 1.1193x over previous
import functools

import jax
import jax.numpy as jnp
from jax import lax
from jax.experimental import pallas as pl
from jax.experimental.pallas import tpu as pltpu
from jax.experimental.pallas import tpu_sc as plsc

N_USER = 100000
N_ITEM = 100000
D = 64
B = 16384

NC = 2
NS = 16
NW = NC * NS
BPW = B // NW
CH = 128
NCH = BPW // CH


@functools.partial(
    pl.kernel,
    out_type=(
        jax.ShapeDtypeStruct((B,), jnp.float32),
        jax.ShapeDtypeStruct((B,), jnp.float32),
    ),
    mesh=plsc.VectorSubcoreMesh(core_axis_name="c", subcore_axis_name="s"),
    compiler_params=pltpu.CompilerParams(needs_layout_passes=False),
    scratch_types=[
        pltpu.VMEM((BPW,), jnp.int32),
        pltpu.VMEM((BPW,), jnp.int32),
        pltpu.VMEM((BPW,), jnp.int32),
        pltpu.VMEM((CH, 2 * D), jnp.float32),
        pltpu.VMEM((CH, 2 * D), jnp.float32),
        pltpu.VMEM((CH, 2 * D), jnp.float32),
        pltpu.VMEM((BPW,), jnp.float32),
        pltpu.VMEM((BPW,), jnp.float32),
        pltpu.VMEM((16 * 17,), jnp.float32),
        pltpu.VMEM((16 * 17,), jnp.float32),
        pltpu.SemaphoreType.DMA,
        pltpu.SemaphoreType.DMA,
    ],
)
def _mcbpr_sc(u_hbm, i_hbm, j_hbm, eu_hbm, ei_hbm, oi_hbm, oj_hbm,
              u_v, i_v, j_v, ur_v, ir_v, jr_v,
              oi_v, oj_v, pi_v, pj_v, sem, isem):
    wid = lax.axis_index("s") * NC + lax.axis_index("c")
    base = wid * BPW

    idx_copies = [
        pltpu.async_copy(u_hbm.at[pl.ds(base, BPW)], u_v, isem),
        pltpu.async_copy(i_hbm.at[pl.ds(base, BPW)], i_v, isem),
        pltpu.async_copy(j_hbm.at[pl.ds(base, BPW)], j_v, isem),
    ]
    for c in idx_copies:
        c.wait()

    lanes = lax.iota(jnp.int32, 16)
    zero = jnp.zeros((16,), jnp.float32)
    tidx = lanes * 17

    def chunk_body(c, carry):
        cbase = c * CH
        copies = [
            pltpu.async_copy(eu_hbm.at[u_v.at[pl.ds(cbase, CH)]], ur_v, sem),
            pltpu.async_copy(ei_hbm.at[i_v.at[pl.ds(cbase, CH)]], ir_v, sem),
            pltpu.async_copy(ei_hbm.at[j_v.at[pl.ds(cbase, CH)]], jr_v, sem),
        ]
        for cp in copies:
            cp.wait()

        def group_body(g, gcarry):
            for r in range(16):
                row = g * 16 + r
                u0 = ur_v[row, pl.ds(0, 16)]
                u1 = ur_v[row, pl.ds(16, 16)]
                u2 = ur_v[row, pl.ds(32, 16)]
                u3 = ur_v[row, pl.ds(48, 16)]
                pi = (u0 * ir_v[row, pl.ds(0, 16)]
                      + u1 * ir_v[row, pl.ds(16, 16)]
                      + u2 * ir_v[row, pl.ds(32, 16)]
                      + u3 * ir_v[row, pl.ds(48, 16)])
                pj = (u0 * jr_v[row, pl.ds(0, 16)]
                      + u1 * jr_v[row, pl.ds(16, 16)]
                      + u2 * jr_v[row, pl.ds(32, 16)]
                      + u3 * jr_v[row, pl.ds(48, 16)])
                pi_v[pl.ds(r * 17, 16)] = pi
                pj_v[pl.ds(r * 17, 16)] = pj
            ai = zero
            aj = zero
            for col in range(16):
                ai = ai + plsc.load_gather(pi_v, [tidx + col])
                aj = aj + plsc.load_gather(pj_v, [tidx + col])
            gsl = pl.ds(cbase + g * 16, 16)
            oi_v[gsl] = ai
            oj_v[gsl] = aj
            return gcarry

        lax.fori_loop(0, CH // 16, group_body, 0)
        return carry

    lax.fori_loop(0, NCH, chunk_body, 0)

    pltpu.sync_copy(oi_v, oi_hbm.at[pl.ds(base, BPW)])
    pltpu.sync_copy(oj_v, oj_hbm.at[pl.ds(base, BPW)])


def kernel(u, i, j, embed_user, embed_item):
    pad = ((0, 0), (0, D))
    return _mcbpr_sc(u.astype(jnp.int32), i.astype(jnp.int32),
                     j.astype(jnp.int32),
                     jnp.pad(embed_user, pad),
                     jnp.pad(embed_item, pad))
